# Initial kernel scaffold; baseline (speedup 1.0000x reference)
import jax
import jax.numpy as jnp
from jax.experimental import pallas as pl


def kernel(loc_data, conf_data, priors, targets):
    raise NotImplementedError("write your pallas kernel here")


# speedup vs baseline: 22.0641x; 22.0641x over previous
import functools

import jax
import jax.numpy as jnp
from jax import lax
from jax.experimental import pallas as pl
from jax.experimental.pallas import tpu as pltpu

_NUM_CLASSES = 8
_THRESHOLD = 0.5
_NEGPOS_RATIO = 3
_VAR0 = 0.1
_VAR1 = 0.2
_LANES = 128


def _mbl_body(tgt_ref, loc_ref, conf_ref, pri_ref, out_ref, *, n_obj, n_priors,
              rows, n_cls):
    R = rows
    img = pl.program_id(0)

    row_i = lax.broadcasted_iota(jnp.int32, (R, _LANES), 0)
    col_i = lax.broadcasted_iota(jnp.int32, (R, _LANES), 1)
    flat = row_i * _LANES + col_i
    valid = flat < n_priors

    pri = pri_ref[...]
    pcx = pri[0 * R:1 * R]
    pcy = pri[1 * R:2 * R]
    pw = pri[2 * R:3 * R]
    ph = pri[3 * R:4 * R]
    px1 = pcx - pw / 2.0
    py1 = pcy - ph / 2.0
    px2 = pcx + pw / 2.0
    py2 = pcy + ph / 2.0
    parea = (px2 - px1) * (py2 - py1)

    loc = loc_ref[0]
    lx = loc[0 * R:1 * R]
    ly = loc[1 * R:2 * R]
    lw = loc[2 * R:3 * R]
    lh = loc[3 * R:4 * R]

    ts = [[tgt_ref[img, f, j] for f in range(5)] for j in range(n_obj)]
    tareas = [(t[2] - t[0]) * (t[3] - t[1]) for t in ts]

    bto = None
    bti = None
    bpi = []
    BIG = jnp.int32(1 << 30)
    for j in range(n_obj):
        ax1, ay1, ax2, ay2, _ = ts[j]
        iw = jnp.maximum(jnp.minimum(ax2, px2) - jnp.maximum(ax1, px1), 0.0)
        ih = jnp.maximum(jnp.minimum(ay2, py2) - jnp.maximum(ay1, py1), 0.0)
        inter = iw * ih
        iou = inter / (tareas[j] + parea - inter)
        iou = jnp.where(valid, iou, -1.0)
        m = jnp.max(iou)
        bpi.append(jnp.min(jnp.where(iou == m, flat, BIG)))
        if j == 0:
            bto = iou
            bti = jnp.zeros((R, _LANES), jnp.int32)
        else:
            bti = jnp.where(iou > bto, j, bti)
            bto = jnp.maximum(bto, iou)

    for j in range(n_obj):
        hit = flat == bpi[j]
        bto = jnp.where(hit, 2.0, bto)
        bti = jnp.where(hit, j, bti)

    pos = bto >= _THRESHOLD
    num_pos = jnp.sum(pos.astype(jnp.int32))

    mx1 = jnp.full((R, _LANES), ts[0][0])
    my1 = jnp.full((R, _LANES), ts[0][1])
    mx2 = jnp.full((R, _LANES), ts[0][2])
    my2 = jnp.full((R, _LANES), ts[0][3])
    mlab = jnp.full((R, _LANES), ts[0][4])
    for j in range(1, n_obj):
        s = bti == j
        mx1 = jnp.where(s, ts[j][0], mx1)
        my1 = jnp.where(s, ts[j][1], my1)
        mx2 = jnp.where(s, ts[j][2], mx2)
        my2 = jnp.where(s, ts[j][3], my2)
        mlab = jnp.where(s, ts[j][4], mlab)

    ecx = ((mx1 + mx2) / 2.0 - pcx) / (_VAR0 * pw)
    ecy = ((my1 + my2) / 2.0 - pcy) / (_VAR0 * ph)
    ew = jnp.log((mx2 - mx1) / pw) / _VAR1
    eh = jnp.log((my2 - my1) / ph) / _VAR1

    def sl1(d):
        ad = jnp.abs(d)
        return jnp.where(ad < 1.0, 0.5 * d * d, ad - 0.5)

    ll = jnp.sum(jnp.where(
        pos, sl1(lx - ecx) + sl1(ly - ecy) + sl1(lw - ew) + sl1(lh - eh), 0.0))

    dcx = pcx + lx * _VAR0 * pw
    dcy = pcy + ly * _VAR0 * ph
    dw = pw * jnp.exp(lw * _VAR1)
    dh = ph * jnp.exp(lh * _VAR1)
    dx1 = dcx - dw / 2.0
    dy1 = dcy - dh / 2.0
    dx2 = dcx + dw / 2.0
    dy2 = dcy + dh / 2.0
    darea = (dx2 - dx1) * (dy2 - dy1)

    bti1 = jnp.zeros((R, _LANES), jnp.int32)
    bo1 = None
    for j in range(n_obj):
        ax1, ay1, ax2, ay2, _ = ts[j]
        iw = jnp.maximum(jnp.minimum(ax2, dx2) - jnp.maximum(ax1, dx1), 0.0)
        ih = jnp.maximum(jnp.minimum(ay2, dy2) - jnp.maximum(ay1, dy1), 0.0)
        inter = iw * ih
        iou = inter / (tareas[j] + darea - inter)
        if j == 0:
            bo1 = iou
        else:
            bti1 = jnp.where(iou > bo1, j, bti1)
            bo1 = jnp.maximum(bo1, iou)

    gx1 = jnp.full((R, _LANES), ts[0][0])
    gy1 = jnp.full((R, _LANES), ts[0][1])
    gx2 = jnp.full((R, _LANES), ts[0][2])
    gy2 = jnp.full((R, _LANES), ts[0][3])
    for j in range(1, n_obj):
        s = bti1 == j
        gx1 = jnp.where(s, ts[j][0], gx1)
        gy1 = jnp.where(s, ts[j][1], gy1)
        gx2 = jnp.where(s, ts[j][2], gx2)
        gy2 = jnp.where(s, ts[j][3], gy2)

    riw = jnp.maximum(jnp.minimum(gx2, dx2) - jnp.maximum(gx1, dx1), 0.0)
    rih = jnp.maximum(jnp.minimum(gy2, dy2) - jnp.maximum(gy1, dy1), 0.0)
    iog = riw * rih / ((gx2 - gx1) * (gy2 - gy1))
    arg = jnp.maximum(1.0 - iog, 0.0) + 1e-10
    lr = jnp.sum(jnp.where(pos, -jnp.log(arg), 0.0))

    conf = conf_ref[0]
    cls = [conf[c * R:(c + 1) * R] for c in range(n_cls)]
    cmax = cls[0]
    for c in range(1, n_cls):
        cmax = jnp.maximum(cmax, cls[c])
    ssum = jnp.exp(cls[0] - cmax)
    for c in range(1, n_cls):
        ssum = ssum + jnp.exp(cls[c] - cmax)
    lse = jnp.log(ssum) + cmax
    ct = jnp.where(pos, mlab.astype(jnp.int32) + 1, 0)
    chosen = jnp.where(ct == 0, cls[0], 0.0)
    for c in range(1, n_cls):
        chosen = chosen + jnp.where(ct == c, cls[c], 0.0)
    ce = jnp.where(valid, lse - chosen, 0.0)

    mce = jnp.where(pos, 0.0, ce)
    bits = lax.bitcast_convert_type(mce, jnp.int32)
    k = jnp.minimum(_NEGPOS_RATIO * num_pos, n_priors - 1)

    def bs_body(_, carry):
        lo, hi = carry
        mid = lo + ((hi - lo + 1) >> 1)
        cnt = jnp.sum((bits >= mid).astype(jnp.int32))
        take = cnt >= k
        return (jnp.where(take, mid, lo), jnp.where(take, hi, mid - 1))

    lo, _ = lax.fori_loop(0, 31, bs_body,
                          (jnp.int32(0), jnp.int32(0x7f800000)))
    kth = lax.bitcast_convert_type(lo, jnp.float32)
    gt = bits > lo
    cnt_gt = jnp.sum(gt.astype(jnp.int32))
    lc = (jnp.sum(jnp.where(pos, ce, 0.0))
          + jnp.sum(jnp.where(gt, mce, 0.0))
          + (k - cnt_gt).astype(jnp.float32) * kth)

    r8 = lax.broadcasted_iota(jnp.int32, (8, _LANES), 0)
    acc = (jnp.where(r8 == 0, ll, 0.0) + jnp.where(r8 == 1, lr, 0.0)
           + jnp.where(r8 == 2, lc, 0.0)
           + jnp.where(r8 == 3, num_pos.astype(jnp.float32), 0.0))

    @pl.when(img == 0)
    def _():
        out_ref[...] = jnp.zeros_like(out_ref)

    out_ref[...] += acc


def kernel(loc_data, conf_data, priors, targets):
    B, P, _ = loc_data.shape
    C = conf_data.shape[-1]
    NOBJ = targets.shape[1]
    R = -(-P // _LANES)
    R = -(-R // 8) * 8
    PP = R * _LANES

    locT = jnp.transpose(loc_data, (0, 2, 1))
    locT = jnp.pad(locT, ((0, 0), (0, 0), (0, PP - P))).reshape(B, 4 * R, _LANES)
    confT = jnp.transpose(conf_data, (0, 2, 1))
    confT = jnp.pad(confT, ((0, 0), (0, 0), (0, PP - P))).reshape(B, C * R, _LANES)
    priT = jnp.transpose(priors, (1, 0))
    pad_col = jnp.array([0.5, 0.5, 1.0, 1.0], jnp.float32)[:, None]
    priT = jnp.concatenate(
        [priT, jnp.broadcast_to(pad_col, (4, PP - P))], axis=1).reshape(4 * R, _LANES)
    tgtT = jnp.transpose(targets, (0, 2, 1))

    out = pl.pallas_call(
        functools.partial(_mbl_body, n_obj=NOBJ, n_priors=P, rows=R, n_cls=C),
        grid=(B,),
        in_specs=[
            pl.BlockSpec(memory_space=pltpu.SMEM),
            pl.BlockSpec((1, 4 * R, _LANES), lambda i: (i, 0, 0)),
            pl.BlockSpec((1, C * R, _LANES), lambda i: (i, 0, 0)),
            pl.BlockSpec((4 * R, _LANES), lambda i: (0, 0)),
        ],
        out_specs=pl.BlockSpec((8, _LANES), lambda i: (0, 0)),
        out_shape=jax.ShapeDtypeStruct((8, _LANES), jnp.float32),
    )(tgtT, locT, confT, priT)

    n = out[3, 0]
    return (out[0, 0] / n, out[1, 0] / n, out[2, 0] / n)


# speedup vs baseline: 22.1127x; 1.0022x over previous
import functools

import jax
import jax.numpy as jnp
from jax import lax
from jax.experimental import pallas as pl
from jax.experimental.pallas import tpu as pltpu

_NUM_CLASSES = 8
_THRESHOLD = 0.5
_NEGPOS_RATIO = 3
_VAR0 = 0.1
_VAR1 = 0.2
_LANES = 128
_IPS = 2


def _one_image(ts, loc, conf, pri_env, n_obj, n_priors, rows, n_cls):
    R = rows
    (pcx, pcy, pw, ph, px1, py1, px2, py2, parea, flat) = pri_env

    lx = loc[0 * R:1 * R]
    ly = loc[1 * R:2 * R]
    lw = loc[2 * R:3 * R]
    lh = loc[3 * R:4 * R]

    tareas = [(t[2] - t[0]) * (t[3] - t[1]) for t in ts]

    bto = None
    bti = None
    bpi = []
    BIG = jnp.int32(1 << 30)
    for j in range(n_obj):
        ax1, ay1, ax2, ay2, _ = ts[j]
        iw = jnp.maximum(jnp.minimum(ax2, px2) - jnp.maximum(ax1, px1), 0.0)
        ih = jnp.maximum(jnp.minimum(ay2, py2) - jnp.maximum(ay1, py1), 0.0)
        inter = iw * ih
        iou = inter / (tareas[j] + parea - inter)
        m = jnp.max(iou)
        bpi.append(jnp.min(jnp.where(iou == m, flat, BIG)))
        if j == 0:
            bto = iou
            bti = jnp.zeros((R, _LANES), jnp.int32)
        else:
            bti = jnp.where(iou > bto, j, bti)
            bto = jnp.maximum(bto, iou)

    for j in range(n_obj):
        hit = flat == bpi[j]
        bto = jnp.where(hit, 2.0, bto)
        bti = jnp.where(hit, j, bti)

    pos = bto >= _THRESHOLD
    num_pos = jnp.sum(pos.astype(jnp.int32))

    mx1 = jnp.full((R, _LANES), ts[0][0])
    my1 = jnp.full((R, _LANES), ts[0][1])
    mx2 = jnp.full((R, _LANES), ts[0][2])
    my2 = jnp.full((R, _LANES), ts[0][3])
    mlab = jnp.full((R, _LANES), ts[0][4])
    for j in range(1, n_obj):
        s = bti == j
        mx1 = jnp.where(s, ts[j][0], mx1)
        my1 = jnp.where(s, ts[j][1], my1)
        mx2 = jnp.where(s, ts[j][2], mx2)
        my2 = jnp.where(s, ts[j][3], my2)
        mlab = jnp.where(s, ts[j][4], mlab)

    ecx = ((mx1 + mx2) / 2.0 - pcx) / (_VAR0 * pw)
    ecy = ((my1 + my2) / 2.0 - pcy) / (_VAR0 * ph)
    ew = jnp.log((mx2 - mx1) / pw) / _VAR1
    eh = jnp.log((my2 - my1) / ph) / _VAR1

    def sl1(d):
        ad = jnp.abs(d)
        return jnp.where(ad < 1.0, 0.5 * d * d, ad - 0.5)

    ll = jnp.sum(jnp.where(
        pos, sl1(lx - ecx) + sl1(ly - ecy) + sl1(lw - ew) + sl1(lh - eh), 0.0))

    dcx = pcx + lx * _VAR0 * pw
    dcy = pcy + ly * _VAR0 * ph
    dw = pw * jnp.exp(lw * _VAR1)
    dh = ph * jnp.exp(lh * _VAR1)
    dx1 = dcx - dw / 2.0
    dy1 = dcy - dh / 2.0
    dx2 = dcx + dw / 2.0
    dy2 = dcy + dh / 2.0
    darea = (dx2 - dx1) * (dy2 - dy1)

    bti1 = jnp.zeros((R, _LANES), jnp.int32)
    bo1 = None
    for j in range(n_obj):
        ax1, ay1, ax2, ay2, _ = ts[j]
        iw = jnp.maximum(jnp.minimum(ax2, dx2) - jnp.maximum(ax1, dx1), 0.0)
        ih = jnp.maximum(jnp.minimum(ay2, dy2) - jnp.maximum(ay1, dy1), 0.0)
        inter = iw * ih
        iou = inter / (tareas[j] + darea - inter)
        if j == 0:
            bo1 = iou
        else:
            bti1 = jnp.where(iou > bo1, j, bti1)
            bo1 = jnp.maximum(bo1, iou)

    gx1 = jnp.full((R, _LANES), ts[0][0])
    gy1 = jnp.full((R, _LANES), ts[0][1])
    gx2 = jnp.full((R, _LANES), ts[0][2])
    gy2 = jnp.full((R, _LANES), ts[0][3])
    for j in range(1, n_obj):
        s = bti1 == j
        gx1 = jnp.where(s, ts[j][0], gx1)
        gy1 = jnp.where(s, ts[j][1], gy1)
        gx2 = jnp.where(s, ts[j][2], gx2)
        gy2 = jnp.where(s, ts[j][3], gy2)

    riw = jnp.maximum(jnp.minimum(gx2, dx2) - jnp.maximum(gx1, dx1), 0.0)
    rih = jnp.maximum(jnp.minimum(gy2, dy2) - jnp.maximum(gy1, dy1), 0.0)
    iog = riw * rih / ((gx2 - gx1) * (gy2 - gy1))
    arg = jnp.maximum(1.0 - iog, 0.0) + 1e-10
    lr = jnp.sum(jnp.where(pos, -jnp.log(arg), 0.0))

    cls = [conf[c * R:(c + 1) * R] for c in range(n_cls)]
    cmax = cls[0]
    for c in range(1, n_cls):
        cmax = jnp.maximum(cmax, cls[c])
    ssum = jnp.exp(cls[0] - cmax)
    for c in range(1, n_cls):
        ssum = ssum + jnp.exp(cls[c] - cmax)
    lse = jnp.log(ssum) + cmax
    ct = jnp.where(pos, mlab.astype(jnp.int32) + 1, 0)
    chosen = jnp.where(ct == 0, cls[0], 0.0)
    for c in range(1, n_cls):
        chosen = chosen + jnp.where(ct == c, cls[c], 0.0)
    ce = lse - chosen

    mce = jnp.where(pos, 0.0, ce)
    bits = lax.bitcast_convert_type(mce, jnp.int32)
    k = jnp.minimum(_NEGPOS_RATIO * num_pos, n_priors - 1)

    def bs_body(_, carry):
        lo, hi = carry
        mid = lo + ((hi - lo + 1) >> 1)
        cnt = jnp.sum((bits >= mid).astype(jnp.int32))
        take = cnt >= k
        return (jnp.where(take, mid, lo), jnp.where(take, hi, mid - 1))

    lo, _ = lax.fori_loop(0, 31, bs_body,
                          (jnp.int32(0), jnp.int32(0x7f800000)))
    kth = lax.bitcast_convert_type(lo, jnp.float32)
    gt = bits > lo
    cnt_gt = jnp.sum(gt.astype(jnp.int32))
    lc = (jnp.sum(jnp.where(pos, ce, 0.0))
          + jnp.sum(jnp.where(gt, mce, 0.0))
          + (k - cnt_gt).astype(jnp.float32) * kth)

    return ll, lr, lc, num_pos.astype(jnp.float32)


def _mbl_body(tgt_ref, loc_ref, conf_ref, pri_ref, out_ref, *, n_obj, n_priors,
              rows, n_cls, ips):
    R = rows
    step = pl.program_id(0)

    row_i = lax.broadcasted_iota(jnp.int32, (R, _LANES), 0)
    col_i = lax.broadcasted_iota(jnp.int32, (R, _LANES), 1)
    flat = row_i * _LANES + col_i

    pri = pri_ref[...]
    pcx = pri[0 * R:1 * R]
    pcy = pri[1 * R:2 * R]
    pw = pri[2 * R:3 * R]
    ph = pri[3 * R:4 * R]
    px1 = pcx - pw / 2.0
    py1 = pcy - ph / 2.0
    px2 = pcx + pw / 2.0
    py2 = pcy + ph / 2.0
    parea = (px2 - px1) * (py2 - py1)
    pri_env = (pcx, pcy, pw, ph, px1, py1, px2, py2, parea, flat)

    ll = jnp.float32(0.0)
    lr = jnp.float32(0.0)
    lc = jnp.float32(0.0)
    npos = jnp.float32(0.0)
    for s in range(ips):
        img = step * ips + s
        ts = [[tgt_ref[img, f, j] for f in range(5)] for j in range(n_obj)]
        a, b, c, d = _one_image(ts, loc_ref[s], conf_ref[s], pri_env,
                                n_obj, n_priors, R, n_cls)
        ll += a
        lr += b
        lc += c
        npos += d

    r8 = lax.broadcasted_iota(jnp.int32, (8, _LANES), 0)
    acc = (jnp.where(r8 == 0, ll, 0.0) + jnp.where(r8 == 1, lr, 0.0)
           + jnp.where(r8 == 2, lc, 0.0) + jnp.where(r8 == 3, npos, 0.0))

    @pl.when(step == 0)
    def _():
        out_ref[...] = jnp.zeros_like(out_ref)

    out_ref[...] += acc


def kernel(loc_data, conf_data, priors, targets):
    B, P, _ = loc_data.shape
    C = conf_data.shape[-1]
    NOBJ = targets.shape[1]
    R = -(-P // _LANES)
    R = -(-R // 8) * 8
    PP = R * _LANES

    locT = jnp.transpose(loc_data, (0, 2, 1))
    locT = jnp.pad(locT, ((0, 0), (0, 0), (0, PP - P))).reshape(B, 4 * R, _LANES)
    confT = jnp.transpose(conf_data, (0, 2, 1))
    pad_cls = jnp.concatenate(
        [jnp.zeros((1, 1), jnp.float32), jnp.full((C - 1, 1), -30.0)], axis=0)
    confT = jnp.concatenate(
        [confT, jnp.broadcast_to(pad_cls, (B, C, PP - P))], axis=2)
    confT = confT.reshape(B, C * R, _LANES)
    priT = jnp.transpose(priors, (1, 0))
    pad_col = jnp.array([10.5, 10.5, 1.0, 1.0], jnp.float32)[:, None]
    priT = jnp.concatenate(
        [priT, jnp.broadcast_to(pad_col, (4, PP - P))], axis=1).reshape(4 * R, _LANES)
    tgtT = jnp.transpose(targets, (0, 2, 1))

    out = pl.pallas_call(
        functools.partial(_mbl_body, n_obj=NOBJ, n_priors=P, rows=R, n_cls=C,
                          ips=_IPS),
        grid=(B // _IPS,),
        in_specs=[
            pl.BlockSpec(memory_space=pltpu.SMEM),
            pl.BlockSpec((_IPS, 4 * R, _LANES), lambda i: (i, 0, 0)),
            pl.BlockSpec((_IPS, C * R, _LANES), lambda i: (i, 0, 0)),
            pl.BlockSpec((4 * R, _LANES), lambda i: (0, 0)),
        ],
        out_specs=pl.BlockSpec((8, _LANES), lambda i: (0, 0)),
        out_shape=jax.ShapeDtypeStruct((8, _LANES), jnp.float32),
    )(tgtT, locT, confT, priT)

    n = out[3, 0]
    return (out[0, 0] / n, out[1, 0] / n, out[2, 0] / n)
